# baseline (speedup 1.0000x reference)
import functools

import jax
import jax.numpy as jnp
from jax import lax
from jax.experimental import pallas as pl
from jax.experimental.pallas import tpu as pltpu
from jax.experimental.pallas import tpu_sc as plsc

B = 16384
D = 32
NC, NS = 2, 16
NW = NC * NS
W = 1024

VU = 1000001
RPW_U = 31744
NCH_U = RPW_U // W
M_U = 999424
TW_U = 640

VM = 100001
RPW_M = 4096
NCH_M = RPW_M // W
M_M = 99328
TW_M = 768

CAP = 16400
TRASH = 16384
NBUF = 4

_mesh = plsc.VectorSubcoreMesh(core_axis_name="c", subcore_axis_name="s")


def _srl(x, k):
    return lax.shift_right_logical(x, jnp.int32(k))


@functools.partial(
    pl.kernel,
    out_type=(
        jax.ShapeDtypeStruct((B + 8, 128), jnp.float32),
        jax.ShapeDtypeStruct((B + 8, 128), jnp.float32),
    ),
    mesh=_mesh,
    compiler_params=pltpu.CompilerParams(
        use_tc_tiling_on_sc=True, needs_layout_passes=False),
    scratch_types=[
        pltpu.VMEM((B,), jnp.int32),
        pltpu.VMEM((B,), jnp.int32),
        pltpu.VMEM((CAP,), jnp.int32),
        pltpu.VMEM((CAP,), jnp.int32),
        pltpu.VMEM((32, W), jnp.float32),
        pltpu.VMEM((NBUF * 24, 128), jnp.float32),
        pltpu.SemaphoreType.DMA,
        pltpu.SemaphoreType.DMA,
        pltpu.SemaphoreType.DMA,
        pltpu.SemaphoreType.DMA,
    ],
)
def _sc_gather(uid_hbm, mid_hbm, ut_hbm, mt_hbm, tu_hbm, tm_hbm,
               ue_hbm, me_hbm,
               uid_v, mid_v, cv_v, cb_v, slab, stage,
               sem0, sem1, sem2, sem3):
    wid = lax.axis_index("s") * NC + lax.axis_index("c")
    lanes = lax.iota(jnp.int32, 16)
    sems = (sem0, sem1, sem2, sem3)

    pltpu.sync_copy(uid_hbm, uid_v)
    pltpu.sync_copy(mid_hbm, mid_v)

    def run_table(idx_v, tab_hbm, tail_hbm, out_hbm, rpw, nch, m_end, tw):
        lo = wid * rpw
        hi = lo + rpw

        def init_body(i, _):
            cv_v[pl.ds(i * 16, 16)] = jnp.full((16,), -1, jnp.int32)
            cb_v[pl.ds(i * 16, 16)] = jnp.full((16,), B, jnp.int32)
            return 0
        lax.fori_loop(0, CAP // 16, init_body, 0)

        def scan_body(i, cntv):
            v16 = idx_v[pl.ds(i * 16, 16)]
            ind = (1 - _srl(v16 - lo, 31)) * _srl(v16 - hi, 31)
            pos = ind * (cntv * 16 + lanes) + (1 - ind) * TRASH
            plsc.store_scatter(cv_v, [pos], v16)
            plsc.store_scatter(cb_v, [pos], lanes + i * 16)
            return cntv + ind

        cntv = lax.fori_loop(0, B // 16, scan_body,
                             jnp.full((16,), 0, jnp.int32))
        maxc = cntv[0]
        for i in range(1, 16):
            maxc = jnp.maximum(maxc, cntv[i])

        def chunk_body(c, n):
            c0 = pl.multiple_of(lo + c * W, 128)

            @pl.when(c0 < m_end)
            def _fetch_main():
                pltpu.sync_copy(tab_hbm.at[:, pl.ds(c0, W)], slab)

            @pl.when(c0 == m_end)
            def _fetch_tail():
                pltpu.sync_copy(tail_hbm, slab.at[:, pl.ds(0, tw)])

            def group_body(g, n):
                v16 = cv_v[pl.ds(g * 16, 16)]
                inm = (v16 >= c0) & (v16 < c0 + W)
                nhit = plsc.all_reduce_population_count(inm)

                def do_hits(n):
                    ind = (1 - _srl(v16 - c0, 31)) * _srl(v16 - (c0 + W), 31)
                    vloc = ind * (v16 - c0)
                    k = lax.rem(n, jnp.int32(NBUF))
                    rb = pl.multiple_of(k * 24, 8)
                    row = rb + ind * lanes + (1 - ind) * 16
                    for kk in range(NBUF):
                        @pl.when((k == kk) & (n >= NBUF))
                        def _drain():
                            pltpu.make_async_copy(
                                out_hbm.at[pl.ds(0, 16)],
                                stage.at[pl.ds(kk * 24, 16)],
                                sems[kk]).wait()
                    for d in range(D):
                        didx = jnp.full((16,), d, jnp.int32)
                        x = plsc.load_gather(slab, [didx, vloc])
                        plsc.store_scatter(stage, [row, didx], x)
                    b16 = cb_v[pl.ds(g * 16, 16)]
                    bidx = ind * b16 + (1 - ind) * B
                    for kk in range(NBUF):
                        @pl.when(k == kk)
                        def _issue():
                            pltpu.async_copy(
                                stage.at[pl.ds(kk * 24, 16)],
                                out_hbm.at[bidx], sems[kk])
                    return n + 1

                return lax.cond(nhit[0] > 0, do_hits, lambda n: n, n)

            return lax.fori_loop(0, maxc, group_body, n)

        n_final = lax.fori_loop(0, nch, chunk_body, jnp.int32(0))
        for kk in range(NBUF):
            @pl.when(n_final > kk)
            def _drain_tail():
                pltpu.make_async_copy(
                    out_hbm.at[pl.ds(0, 16)],
                    stage.at[pl.ds(kk * 24, 16)],
                    sems[kk]).wait()

    run_table(uid_v, ut_hbm, tu_hbm, ue_hbm, RPW_U, NCH_U, M_U, TW_U)
    run_table(mid_v, mt_hbm, tm_hbm, me_hbm, RPW_M, NCH_M, M_M, TW_M)


BLK = 2048


def _mlp_body(ue, me, w1u, w1m, b1, w2, b2, w3t, b3, out):
    ue32 = ue[...][:, :D]
    me32 = me[...][:, :D]
    h = jnp.dot(ue32, w1u[...], preferred_element_type=jnp.float32)
    h = h + jnp.dot(me32, w1m[...], preferred_element_type=jnp.float32)
    h = jnp.maximum(h + b1[...], 0.0)
    h = jnp.dot(h, w2[...], preferred_element_type=jnp.float32) + b2[...]
    h = jnp.maximum(h, 0.0)
    out[...] = jnp.sum(h * w3t[...], axis=1, keepdims=True) + b3[...]


_mlp = pl.pallas_call(
    _mlp_body,
    grid=(B // BLK,),
    in_specs=[
        pl.BlockSpec((BLK, 128), lambda i: (i, 0)),
        pl.BlockSpec((BLK, 128), lambda i: (i, 0)),
        pl.BlockSpec((D, 256), lambda i: (0, 0)),
        pl.BlockSpec((D, 256), lambda i: (0, 0)),
        pl.BlockSpec((1, 256), lambda i: (0, 0)),
        pl.BlockSpec((256, 64), lambda i: (0, 0)),
        pl.BlockSpec((1, 64), lambda i: (0, 0)),
        pl.BlockSpec((1, 64), lambda i: (0, 0)),
        pl.BlockSpec((1, 1), lambda i: (0, 0)),
    ],
    out_specs=pl.BlockSpec((BLK, 1), lambda i: (i, 0)),
    out_shape=jax.ShapeDtypeStruct((B, 1), jnp.float32),
)


def kernel(user_id, movie_title, user_table, movie_table, W1, b1, W2, b2, W3, b3):
    uid = user_id.astype(jnp.int32)
    mid = movie_title.astype(jnp.int32)
    tail_u = jnp.pad(user_table[M_U:].T, ((0, 0), (0, TW_U - (VU - M_U))))
    tail_m = jnp.pad(movie_table[M_M:].T, ((0, 0), (0, TW_M - (VM - M_M))))
    ue_p, me_p = _sc_gather(uid, mid, user_table.T, movie_table.T,
                            tail_u, tail_m)
    return _mlp(
        ue_p,
        me_p,
        W1[:D],
        W1[D:],
        b1.reshape(1, 256),
        W2,
        b2.reshape(1, 64),
        W3.reshape(1, 64),
        b3.reshape(1, 1),
    )


# speedup vs baseline: 1.4765x; 1.4765x over previous
import functools

import jax
import jax.numpy as jnp
from jax import lax
from jax.experimental import pallas as pl
from jax.experimental.pallas import tpu as pltpu
from jax.experimental.pallas import tpu_sc as plsc

B = 16384
D = 32
NC, NS = 2, 16
W = 512

VU = 1000001
RPW_U = 31744
NCH_U = RPW_U // W
M_U = 999936
TW_U = 128

VM = 100001
RPW_M = 3584
NCH_M = RPW_M // W
M_M = 99840
TW_M = 256

CAP = 16400
TRASH = 16384
PL = 64
NROW = 16 * PL
TOTROW = NROW + 16
NWORD = NROW * D

_mesh = plsc.VectorSubcoreMesh(core_axis_name="c", subcore_axis_name="s")


def _srl(x, k):
    return lax.shift_right_logical(x, jnp.int32(k))


@functools.partial(
    pl.kernel,
    out_type=(
        jax.ShapeDtypeStruct((B * D + NWORD,), jnp.float32),
        jax.ShapeDtypeStruct((B * D + NWORD,), jnp.float32),
    ),
    mesh=_mesh,
    compiler_params=pltpu.CompilerParams(
        use_tc_tiling_on_sc=True, needs_layout_passes=False),
    scratch_types=[
        pltpu.VMEM((2048,), jnp.int32),
        pltpu.VMEM((CAP,), jnp.int32),
        pltpu.VMEM((CAP,), jnp.int32),
        pltpu.VMEM((32, W), jnp.float32),
        pltpu.VMEM((TOTROW * D,), jnp.float32),
        pltpu.VMEM((TOTROW,), jnp.int32),
        pltpu.VMEM((NWORD,), jnp.int32),
        pltpu.SemaphoreType.DMA,
    ],
)
def _sc_gather(uid_hbm, mid_hbm, ut_hbm, mt_hbm, tu_hbm, tm_hbm,
               ue_hbm, me_hbm,
               piece, cv_v, cb_v, slab, stage, bvals, widx, sem):
    wid = lax.axis_index("s") * NC + lax.axis_index("c")
    lanes = lax.iota(jnp.int32, 16)

    def run_table(idx_hbm, tab_hbm, tail_hbm, out_hbm, rpw, nch, m_end, tw):
        lo = wid * rpw
        hi = lo + rpw

        def init_body(i, _):
            cv_v[pl.ds(i * 16, 16)] = jnp.full((16,), -1, jnp.int32)
            cb_v[pl.ds(i * 16, 16)] = jnp.full((16,), B, jnp.int32)
            return 0
        lax.fori_loop(0, CAP // 16, init_body, 0)

        def piece_body(p, cntv):
            pltpu.sync_copy(idx_hbm.at[pl.ds(p * 2048, 2048)], piece)

            def scan_body(i, cntv):
                v16 = piece[pl.ds(i * 16, 16)]
                ind = (1 - _srl(v16 - lo, 31)) * _srl(v16 - hi, 31)
                pos = ind * (cntv * 16 + lanes) + (1 - ind) * TRASH
                plsc.store_scatter(cv_v, [pos], v16)
                plsc.store_scatter(cb_v, [pos], lanes + (p * 2048 + i * 16))
                return cntv + ind

            return lax.fori_loop(0, 2048 // 16, scan_body, cntv)

        cntv = lax.fori_loop(0, B // 2048, piece_body,
                             jnp.full((16,), 0, jnp.int32))
        maxc = cntv[0]
        for i in range(1, 16):
            maxc = jnp.maximum(maxc, cntv[i])
        npass = lax.div(maxc + (PL - 1), jnp.int32(PL))

        def pass_body(p, _):
            pbase = p * PL

            def rinit_body(i, _):
                bvals[pl.ds(i * 16, 16)] = jnp.full((16,), B, jnp.int32)
                return 0
            lax.fori_loop(0, NROW // 16, rinit_body, 0)

            def chunk_body(c, cntl):
                c0 = pl.multiple_of(lo + c * W, 128)

                @pl.when(c0 < m_end)
                def _fetch_main():
                    pltpu.sync_copy(tab_hbm.at[:, pl.ds(c0, W)], slab)

                @pl.when(c0 == m_end)
                def _fetch_tail():
                    pltpu.sync_copy(tail_hbm, slab.at[:, pl.ds(0, tw)])

                def group_body(g, cntl):
                    v16 = cv_v[pl.ds(g * 16, 16)]
                    inm = (v16 >= c0) & (v16 < c0 + W)
                    nhit = plsc.all_reduce_population_count(inm)

                    def do_hits(cntl):
                        ind = (1 - _srl(v16 - c0, 31)) * _srl(
                            v16 - (c0 + W), 31)
                        vloc = ind * (v16 - c0)
                        inp = (1 - _srl(cntl - pbase, 31)) * _srl(
                            cntl - (pbase + PL), 31)
                        eff = ind * inp
                        row = (eff * (lanes * PL + (cntl - pbase))
                               + (1 - eff) * NROW)
                        rowb = row * D
                        for d in range(D):
                            didx = jnp.full((16,), d, jnp.int32)
                            x = plsc.load_gather(slab, [didx, vloc])
                            plsc.store_scatter(stage, [rowb + d], x)
                        b16 = cb_v[pl.ds(g * 16, 16)]
                        plsc.store_scatter(bvals, [row], b16)
                        return cntl + ind

                    return lax.cond(nhit[0] > 0, do_hits, lambda v: v, cntl)

                return lax.fori_loop(0, maxc, group_body, cntl)

            lax.fori_loop(0, nch, chunk_body, jnp.full((16,), 0, jnp.int32))

            def widx_body(j, _):
                b16 = bvals[pl.ds(j * 16, 16)]
                isd = 1 - _srl(b16 - B, 31)
                for d in range(D):
                    a16 = (j * 16 + lanes) * D + d
                    wdst = (1 - isd) * (b16 * D + d) + isd * (B * D + a16)
                    plsc.store_scatter(widx, [a16], wdst)
                return 0
            lax.fori_loop(0, NROW // 16, widx_body, 0)

            pltpu.async_copy(stage.at[pl.ds(0, NWORD)], out_hbm.at[widx],
                             sem).wait()
            return 0

        lax.fori_loop(0, npass, pass_body, 0)

    run_table(uid_hbm, ut_hbm, tu_hbm, ue_hbm, RPW_U, NCH_U, M_U, TW_U)
    run_table(mid_hbm, mt_hbm, tm_hbm, me_hbm, RPW_M, NCH_M, M_M, TW_M)


BLK = 2048


def _mlp_body(ue, me, w1u, w1m, b1, w2, b2, w3t, b3, out):
    h = jnp.dot(ue[...], w1u[...], preferred_element_type=jnp.float32)
    h = h + jnp.dot(me[...], w1m[...], preferred_element_type=jnp.float32)
    h = jnp.maximum(h + b1[...], 0.0)
    h = jnp.dot(h, w2[...], preferred_element_type=jnp.float32) + b2[...]
    h = jnp.maximum(h, 0.0)
    out[...] = jnp.sum(h * w3t[...], axis=1, keepdims=True) + b3[...]


_mlp = pl.pallas_call(
    _mlp_body,
    grid=(B // BLK,),
    in_specs=[
        pl.BlockSpec((BLK, D), lambda i: (i, 0)),
        pl.BlockSpec((BLK, D), lambda i: (i, 0)),
        pl.BlockSpec((D, 256), lambda i: (0, 0)),
        pl.BlockSpec((D, 256), lambda i: (0, 0)),
        pl.BlockSpec((1, 256), lambda i: (0, 0)),
        pl.BlockSpec((256, 64), lambda i: (0, 0)),
        pl.BlockSpec((1, 64), lambda i: (0, 0)),
        pl.BlockSpec((1, 64), lambda i: (0, 0)),
        pl.BlockSpec((1, 1), lambda i: (0, 0)),
    ],
    out_specs=pl.BlockSpec((BLK, 1), lambda i: (i, 0)),
    out_shape=jax.ShapeDtypeStruct((B, 1), jnp.float32),
)


def kernel(user_id, movie_title, user_table, movie_table, W1, b1, W2, b2, W3, b3):
    uid = user_id.astype(jnp.int32)
    mid = movie_title.astype(jnp.int32)
    tail_u = jnp.pad(user_table[M_U:].T, ((0, 0), (0, TW_U - (VU - M_U))))
    tail_m = jnp.pad(movie_table[M_M:].T, ((0, 0), (0, TW_M - (VM - M_M))))
    ue_f, me_f = _sc_gather(uid, mid, user_table.T, movie_table.T,
                            tail_u, tail_m)
    return _mlp(
        ue_f[:B * D].reshape(B, D),
        me_f[:B * D].reshape(B, D),
        W1[:D],
        W1[D:],
        b1.reshape(1, 256),
        W2,
        b2.reshape(1, 64),
        W3.reshape(1, 64),
        b3.reshape(1, 1),
    )
